# baseline (speedup 1.0000x reference)
import functools

import jax
import jax.numpy as jnp
from jax import lax
from jax.experimental import pallas as pl
from jax.experimental.pallas import tpu as pltpu
from jax.experimental.pallas import tpu_sc as plsc

L = 16
NC = 2
NS = 16
NW = NC * NS

E_TOTAL = 320000
E_PER_W = E_TOTAL // NW
CH = 80
N_CH = E_PER_W // CH
DEPTH = 3
G_PER_CH = CH // L

_BITREV = (0, 8, 4, 12, 2, 10, 6, 14, 1, 9, 5, 13, 3, 11, 7, 15)

_SHUFFLE_DNUMS = lax.GatherDimensionNumbers(
    offset_dims=(), collapsed_slice_dims=(0,), start_index_map=(0,))


def _lane_shuffle(v, perm):
    return lax.gather(v, perm[:, None], _SHUFFLE_DNUMS, slice_sizes=(1,),
                      mode=lax.GatherScatterMode.PROMISE_IN_BOUNDS)


def _sc_kernel(z_hbm, src_hbm, dst_hbm, out_hbm, sidx, didx, srows, drows,
               obuf, pbuf, *sems):
    wid = lax.axis_index("s") * NC + lax.axis_index("c")
    w_base = wid * E_PER_W
    lane = lax.iota(jnp.int32, L)

    pltpu.sync_copy(src_hbm.at[pl.ds(w_base, E_PER_W)], sidx)
    pltpu.sync_copy(dst_hbm.at[pl.ds(w_base, E_PER_W)], didx)

    def issue(c, slot):
        pltpu.async_copy(z_hbm.at[sidx.at[pl.ds(c * CH, CH)]],
                         srows.at[pl.ds(slot * CH, CH)], sems[2 * slot])
        pltpu.async_copy(z_hbm.at[didx.at[pl.ds(c * CH, CH)]],
                         drows.at[pl.ds(slot * CH, CH)], sems[2 * slot + 1])

    def drain(slot):
        pltpu.make_async_copy(z_hbm.at[sidx.at[pl.ds(0, CH)]],
                              srows.at[pl.ds(slot * CH, CH)],
                              sems[2 * slot]).wait()
        pltpu.make_async_copy(z_hbm.at[didx.at[pl.ds(0, CH)]],
                              drows.at[pl.ds(slot * CH, CH)],
                              sems[2 * slot + 1]).wait()

    def combine(a, b, k):
        m = (lane & k) == 0
        t1 = jnp.where(m, a, b)
        t2 = _lane_shuffle(jnp.where(m, b, a), lane ^ k)
        return t1 + t2

    def compute(c, slot):
        sbase = slot * CH

        def edge_acc(e):
            p = [srows[e, pl.ds(j * L, L)] * drows[e, pl.ds(j * L, L)]
                 for j in range(128 // L)]
            while len(p) > 1:
                p = [p[i] + p[i + 1] for i in range(0, len(p), 2)]
            return p[0]

        @plsc.parallel_loop(0, CH // 2, 1, unroll=2)
        def pair_body(p):
            g = p // (L // 2)
            i = p % (L // 2)
            r = ((i & 1) << 2) | (i & 2) | ((i & 4) >> 2)
            ea = sbase + g * L + r
            pbuf[pl.ds(p * L, L)] = combine(edge_acc(ea), edge_acc(ea + 8), 8)

        @plsc.parallel_loop(0, G_PER_CH, 1)
        def group_body(g):
            accs = [pbuf[pl.ds((g * 8 + i) * L, L)] for i in range(8)]
            for k in (4, 2, 1):
                accs = [combine(accs[i], accs[i + 1], k)
                        for i in range(0, len(accs), 2)]
            res = accs[0]
            obuf[pl.ds(c * CH + g * L, L)] = 1.0 / (1.0 + jnp.exp(-res))

    for r in range(DEPTH):
        issue(r, r)

    def ring_body(q, _):
        for r in range(DEPTH):
            c = q * DEPTH + r
            drain(r)
            compute(c, r)

            @pl.when(c + DEPTH < N_CH)
            def _():
                issue(c + DEPTH, r)
        return 0

    n_main = (N_CH // DEPTH) * DEPTH
    lax.fori_loop(0, n_main // DEPTH, ring_body, 0)
    for t in range(n_main, N_CH):
        drain(t - n_main)
        compute(t, t - n_main)

    pltpu.sync_copy(obuf, out_hbm.at[pl.ds(w_base, E_PER_W)])


@jax.jit
def _decode(z, src, dst):
    mesh = plsc.VectorSubcoreMesh(core_axis_name="c", subcore_axis_name="s")
    run = functools.partial(
        pl.kernel,
        mesh=mesh,
        out_type=jax.ShapeDtypeStruct((E_TOTAL,), jnp.float32),
        scratch_types=[
            pltpu.VMEM((E_PER_W,), jnp.int32),
            pltpu.VMEM((E_PER_W,), jnp.int32),
            pltpu.VMEM((DEPTH * CH, 128), jnp.float32),
            pltpu.VMEM((DEPTH * CH, 128), jnp.float32),
            pltpu.VMEM((E_PER_W,), jnp.float32),
            pltpu.VMEM((CH // 2 * L,), jnp.float32),
        ] + [pltpu.SemaphoreType.DMA] * (2 * DEPTH),
    )(_sc_kernel)
    return run(z, src, dst)


def kernel(z, edge_index):
    ei = edge_index.astype(jnp.int32)
    return _decode(z, ei[0], ei[1])
